# baseline (speedup 1.0000x reference)
import jax
import jax.numpy as jnp
from jax import lax
from jax.experimental import pallas as pl
from jax.experimental.pallas import tpu as pltpu
from jax.experimental.pallas import tpu_sc as plsc

_N_POP = 1000
_EMBED_DIM = 32
_N_AF = 16
_TOTAL_DIM = _EMBED_DIM + _N_AF
_BATCH = 16384

_NC = 1
_NS = 16
_NW = _NC * _NS
_BPW = _BATCH // _NW
_PACK = 128 // _EMBED_DIM
_GRP = 4096
_SUB = _GRP // _PACK
_BLK = 16384


def _sc_gather_body(table_hbm, idx_hbm, out_hbm, idx_v, rows_v, sem):
    wid = lax.axis_index("s") * _NC + lax.axis_index("c")
    base = wid * _BPW
    pltpu.sync_copy(idx_hbm.at[pl.ds(base, _BPW)], idx_v)
    pltpu.async_copy(table_hbm.at[idx_v], rows_v, sem).wait()
    nsub = _NW // (_BATCH // _GRP)
    sub = wid % nsub
    blk = wid // nsub
    r0 = _SUB * blk + (_BPW * sub) % _SUB
    c0 = _EMBED_DIM * ((_BPW * sub) // _SUB)
    pltpu.sync_copy(rows_v, out_hbm.at[pl.ds(r0, _BPW), pl.ds(c0, _EMBED_DIM)])


def _sc_gather_packed(table, idx):
    mesh = plsc.VectorSubcoreMesh(
        core_axis_name="c", subcore_axis_name="s", num_cores=_NC)
    return pl.kernel(
        _sc_gather_body,
        out_type=jax.ShapeDtypeStruct((_BATCH // _PACK, 128), jnp.float32),
        mesh=mesh,
        compiler_params=pltpu.CompilerParams(use_tc_tiling_on_sc=False),
        scratch_types=[
            pltpu.VMEM((_BPW,), jnp.int32),
            pltpu.VMEM((_BPW, _EMBED_DIM), jnp.float32),
            pltpu.SemaphoreType.DMA,
        ],
    )(table, idx)


def _dense_body(e4_ref, at_ref, w1_ref, w2_ref, b_ref, g_ref, beta_ref, out_ref):
    e4 = e4_ref[:]
    parts = [
        lax.dot_general(
            w1_ref[:],
            e4[gi * _SUB:(gi + 1) * _SUB, q * _EMBED_DIM:(q + 1) * _EMBED_DIM],
            (((1,), (1,)), ((), ())),
            preferred_element_type=jnp.float32,
        )
        for gi in range(_BLK // _GRP)
        for q in range(_PACK)
    ]
    h = jnp.concatenate(parts, axis=1)
    h = h + lax.dot_general(
        w2_ref[:], at_ref[:],
        (((1,), (0,)), ((), ())),
        preferred_element_type=jnp.float32,
    )
    bcol = lax.broadcast_in_dim(b_ref[:], (_TOTAL_DIM, 1), (0,))
    gcol = lax.broadcast_in_dim(g_ref[:], (_TOTAL_DIM, 1), (0,))
    betacol = lax.broadcast_in_dim(beta_ref[:], (_TOTAL_DIM, 1), (0,))
    h = h + bcol
    mu = jnp.mean(h, axis=0, keepdims=True)
    xc = h - mu
    var = jnp.mean(xc * xc, axis=0, keepdims=True)
    y = xc * lax.rsqrt(var + 1e-5) * gcol + betacol
    out_ref[:] = jnp.maximum(y, 0.0)


def _dense_t(emb4, afT, W, b, gamma, beta):
    w1 = W[:, :_EMBED_DIM]
    w2 = W[:, _EMBED_DIM:]
    grid = (_BATCH // _BLK,)
    return pl.pallas_call(
        _dense_body,
        grid=grid,
        in_specs=[
            pl.BlockSpec((_BLK // _PACK, 128), lambda i: (i, 0)),
            pl.BlockSpec((_N_AF, _BLK), lambda i: (0, i)),
            pl.BlockSpec((_TOTAL_DIM, _EMBED_DIM), lambda i: (0, 0)),
            pl.BlockSpec((_TOTAL_DIM, _N_AF), lambda i: (0, 0)),
            pl.BlockSpec((_TOTAL_DIM,), lambda i: (0,)),
            pl.BlockSpec((_TOTAL_DIM,), lambda i: (0,)),
            pl.BlockSpec((_TOTAL_DIM,), lambda i: (0,)),
        ],
        out_specs=pl.BlockSpec((_TOTAL_DIM, _BLK), lambda i: (0, i)),
        out_shape=jax.ShapeDtypeStruct((_TOTAL_DIM, _BATCH), jnp.float32),
    )(emb4, afT, w1, w2, b, gamma, beta)


def kernel(population_id, allele_freq_features, table, W, b, gamma, beta):
    emb4 = _sc_gather_packed(table, population_id)
    outT = _dense_t(emb4, allele_freq_features.T, W, b, gamma, beta)
    return outT.T


# speedup vs baseline: 1.0357x; 1.0357x over previous
import jax
import jax.numpy as jnp
from jax import lax
from jax.experimental import pallas as pl
from jax.experimental.pallas import tpu as pltpu
from jax.experimental.pallas import tpu_sc as plsc

_N_POP = 1000
_EMBED_DIM = 32
_N_AF = 16
_TOTAL_DIM = _EMBED_DIM + _N_AF
_BATCH = 16384

_NC = 1
_NS = 16
_NW = _NC * _NS
_BPW = _BATCH // _NW
_PACK = 128 // _EMBED_DIM
_GRP = 4096
_SUB = _GRP // _PACK
_BLK = 8192


def _sc_gather_body(table_hbm, idx_hbm, out_hbm, idx_v, rows_v, sem):
    wid = lax.axis_index("s") * _NC + lax.axis_index("c")
    base = wid * _BPW
    pltpu.sync_copy(idx_hbm.at[pl.ds(base, _BPW)], idx_v)
    pltpu.async_copy(table_hbm.at[idx_v], rows_v, sem).wait()
    nsub = _NW // (_BATCH // _GRP)
    sub = wid % nsub
    blk = wid // nsub
    r0 = _SUB * blk + (_BPW * sub) % _SUB
    c0 = _EMBED_DIM * ((_BPW * sub) // _SUB)
    pltpu.sync_copy(rows_v, out_hbm.at[pl.ds(r0, _BPW), pl.ds(c0, _EMBED_DIM)])


def _sc_gather_packed(table, idx):
    mesh = plsc.VectorSubcoreMesh(
        core_axis_name="c", subcore_axis_name="s", num_cores=_NC)
    return pl.kernel(
        _sc_gather_body,
        out_type=jax.ShapeDtypeStruct((_BATCH // _PACK, 128), jnp.float32),
        mesh=mesh,
        compiler_params=pltpu.CompilerParams(use_tc_tiling_on_sc=False),
        scratch_types=[
            pltpu.VMEM((_BPW,), jnp.int32),
            pltpu.VMEM((_BPW, _EMBED_DIM), jnp.float32),
            pltpu.SemaphoreType.DMA,
        ],
    )(table, idx)


def _dense_body(e4_ref, at_ref, w1_ref, w2_ref, b_ref, g_ref, beta_ref, out_ref):
    e4 = e4_ref[:]
    parts = [
        lax.dot_general(
            w1_ref[:],
            e4[gi * _SUB:(gi + 1) * _SUB, q * _EMBED_DIM:(q + 1) * _EMBED_DIM],
            (((1,), (1,)), ((), ())),
            preferred_element_type=jnp.float32,
        )
        for gi in range(_BLK // _GRP)
        for q in range(_PACK)
    ]
    h = jnp.concatenate(parts, axis=1)
    h = h + lax.dot_general(
        w2_ref[:], at_ref[:],
        (((1,), (0,)), ((), ())),
        preferred_element_type=jnp.float32,
    )
    bcol = lax.broadcast_in_dim(b_ref[:], (_TOTAL_DIM, 1), (0,))
    gcol = lax.broadcast_in_dim(g_ref[:], (_TOTAL_DIM, 1), (0,))
    betacol = lax.broadcast_in_dim(beta_ref[:], (_TOTAL_DIM, 1), (0,))
    h = h + bcol
    mu = jnp.mean(h, axis=0, keepdims=True)
    xc = h - mu
    var = jnp.mean(xc * xc, axis=0, keepdims=True)
    y = xc * lax.rsqrt(var + 1e-5) * gcol + betacol
    out_ref[:] = jnp.maximum(y, 0.0)


def _dense_t(emb4, afT, W, b, gamma, beta):
    w1 = W[:, :_EMBED_DIM]
    w2 = W[:, _EMBED_DIM:]
    grid = (_BATCH // _BLK,)
    return pl.pallas_call(
        _dense_body,
        grid=grid,
        in_specs=[
            pl.BlockSpec((_BLK // _PACK, 128), lambda i: (i, 0)),
            pl.BlockSpec((_N_AF, _BLK), lambda i: (0, i)),
            pl.BlockSpec((_TOTAL_DIM, _EMBED_DIM), lambda i: (0, 0)),
            pl.BlockSpec((_TOTAL_DIM, _N_AF), lambda i: (0, 0)),
            pl.BlockSpec((_TOTAL_DIM,), lambda i: (0,)),
            pl.BlockSpec((_TOTAL_DIM,), lambda i: (0,)),
            pl.BlockSpec((_TOTAL_DIM,), lambda i: (0,)),
        ],
        out_specs=pl.BlockSpec((_TOTAL_DIM, _BLK), lambda i: (0, i)),
        out_shape=jax.ShapeDtypeStruct((_TOTAL_DIM, _BATCH), jnp.float32),
    )(emb4, afT, w1, w2, b, gamma, beta)


def kernel(population_id, allele_freq_features, table, W, b, gamma, beta):
    emb4 = _sc_gather_packed(table, population_id)
    outT = _dense_t(emb4, allele_freq_features.T, W, b, gamma, beta)
    return outT.T
